# baseline (speedup 1.0000x reference)
import functools
import math

import jax
import jax.numpy as jnp
from jax import lax
from jax.experimental import pallas as pl
from jax.experimental.pallas import tpu as pltpu
from jax.experimental.pallas import tpu_sc as plsc

D_MODEL = 64
SCALE = math.sqrt(D_MODEL)

NUM_CORES = 2
NUM_SUBCORES = 16
LANES = 16
NW = NUM_CORES * NUM_SUBCORES

NBUF = 2
G = 2
QB = 256


@functools.lru_cache(maxsize=None)
def _build_relayout(V: int):
    n_quads = V // QB
    rem = V - n_quads * QB
    per_w = -(-n_quads // NW)

    mesh = plsc.VectorSubcoreMesh(core_axis_name="c", subcore_axis_name="s")

    @functools.partial(
        pl.kernel,
        out_type=jax.ShapeDtypeStruct((V * D_MODEL,), jnp.float32),
        mesh=mesh,
        scratch_types=[
            pltpu.VMEM((D_MODEL, QB), jnp.float32),
            pltpu.VMEM((D_MODEL, QB), jnp.float32),
            pltpu.VMEM((QB * D_MODEL,), jnp.float32),
            pltpu.VMEM((QB * D_MODEL,), jnp.float32),
            pltpu.SemaphoreType.DMA,
            pltpu.SemaphoreType.DMA,
        ],
        compiler_params=pltpu.CompilerParams(needs_layout_passes=False),
    )
    def k(lutT_hbm, tail_hbm, tab_hbm, inb0_v, inb1_v, outb0_v, outb1_v,
          isem, wsem):
        inb_s = [inb0_v, inb1_v]
        outb_s = [outb0_v, outb1_v]
        wid = lax.axis_index("s") * NUM_CORES + lax.axis_index("c")
        iota16 = jax.lax.iota(jnp.int32, LANES)

        def quad(i):
            return wid + NW * i

        def start_in(i, slot):
            pltpu.async_copy(
                lutT_hbm.at[:, pl.ds(quad(i) * QB, QB)], inb_s[slot], isem
            )

        def wait_in(slot):
            pltpu.make_async_copy(
                lutT_hbm.at[:, pl.ds(0, QB)], inb_s[slot], isem
            ).wait()

        def start_out(i, slot):
            pltpu.async_copy(
                outb_s[slot],
                tab_hbm.at[pl.ds(quad(i) * QB * D_MODEL, QB * D_MODEL)],
                wsem,
            )

        def wait_out(slot):
            pltpu.make_async_copy(
                outb_s[slot], tab_hbm.at[pl.ds(0, QB * D_MODEL)], wsem
            ).wait()

        def transpose(slot):
            inb = inb_s[slot]
            outb = outb_s[slot]

            @pl.loop(0, QB // LANES)
            def _(kk):
                tok = iota16 + kk * LANES
                for d in range(LANES):
                    fbase = jnp.bitwise_and(iota16 + d, LANES - 1)
                    for j in range(D_MODEL // LANES):
                        frow = fbase + j * LANES
                        vals = plsc.load_gather(inb, [frow, tok])
                        plsc.store_scatter(outb, [tok * D_MODEL + frow], vals)

        @pl.when(quad(0) < n_quads)
        def _():
            start_in(0, 0)

        @pl.when(quad(1) < n_quads)
        def _():
            start_in(1, 1)

        @pl.loop(0, per_w, step=NBUF)
        def _(i0):
            for b in range(NBUF):
                i = i0 + b

                @pl.when(quad(i) < n_quads)
                def _():
                    @pl.when(i >= NBUF)
                    def _():
                        wait_out(b)

                    wait_in(b)
                    transpose(b)

                    @pl.when(quad(i + NBUF) < n_quads)
                    def _():
                        start_in(i + NBUF, b)

                    start_out(i, b)

        wait_out(0)
        wait_out(1)

        if rem:
            @pl.when(wid == NW - 1)
            def _():
                pltpu.sync_copy(
                    tail_hbm, outb0_v.at[pl.ds(0, rem * D_MODEL)]
                )
                pltpu.sync_copy(
                    outb0_v.at[pl.ds(0, rem * D_MODEL)],
                    tab_hbm.at[pl.ds(n_quads * QB * D_MODEL, rem * D_MODEL)],
                )

    return k


@functools.lru_cache(maxsize=None)
def _build_gather(S0: int, S1: int, V: int):
    lanes_per_w = S0 // NW
    rows_per_g = G * lanes_per_w
    n_steps = S1 // G

    mesh = plsc.VectorSubcoreMesh(core_axis_name="c", subcore_axis_name="s")

    @functools.partial(
        pl.kernel,
        out_type=jax.ShapeDtypeStruct(
            (S1, D_MODEL // 8, S0 // 128, 8, 128), jnp.float32
        ),
        mesh=mesh,
        scratch_types=[
            pltpu.VMEM((S1, lanes_per_w), jnp.int32),
            pltpu.VMEM((rows_per_g,), jnp.int32),
            pltpu.VMEM((rows_per_g,), jnp.int32),
            pltpu.VMEM((NBUF, rows_per_g, D_MODEL), jnp.float32),
            pltpu.VMEM((NBUF, G, D_MODEL // 8, 1, 8, lanes_per_w), jnp.float32),
            pltpu.SemaphoreType.DMA,
            pltpu.SemaphoreType.DMA,
        ],
        compiler_params=pltpu.CompilerParams(
            needs_layout_passes=False, use_tc_tiling_on_sc=False
        ),
    )
    def k(xt_hbm, tab_hbm, out_hbm, idx_v, pb0_v, pb1_v, rows_v, ob_v,
          gsem, wsem):
        pb = [pb0_v, pb1_v]
        wid = lax.axis_index("s") * NUM_CORES + lax.axis_index("c")
        base = wid * lanes_per_w
        pltpu.sync_copy(xt_hbm.at[:, pl.ds(base, lanes_per_w)], idx_v)

        def compute_p(g, slot):
            for rr in range(G):
                for kk in range(lanes_per_w // LANES):
                    sl = pl.ds(kk * LANES, LANES)
                    dsl = pl.ds(rr * lanes_per_w + kk * LANES, LANES)
                    pb[slot][dsl] = idx_v[g * G + rr, sl]

        def start_gather(slot):
            pltpu.async_copy(tab_hbm.at[pb[slot]], rows_v.at[slot], gsem)

        def wait_gather(slot):
            pltpu.make_async_copy(
                tab_hbm.at[pb[slot]], rows_v.at[slot], gsem
            ).wait()

        def start_write(g, slot):
            pltpu.async_copy(
                ob_v.at[slot], out_hbm.at[pl.ds(g * G, G), :, pl.ds(wid, 1)], wsem
            )

        def wait_write(slot):
            pltpu.make_async_copy(
                ob_v.at[slot], out_hbm.at[pl.ds(0, G), :, pl.ds(0, 1)], wsem
            ).wait()

        iota16 = jax.lax.iota(jnp.int32, LANES)

        def compute_out(g, slot):
            ob = ob_v.at[slot]
            rows = rows_v.at[slot]

            @pl.loop(0, rows_per_g // LANES)
            def _(kk):
                tok = iota16 + kk * LANES
                rr = lax.shift_right_logical(kk, 3)
                kl = lax.bitwise_and(kk, 7)
                rrv = jnp.full((LANES,), rr, jnp.int32)
                tokl = iota16 + kl * LANES
                for d in range(LANES):
                    fbase = jnp.bitwise_and(iota16 + d, LANES - 1)
                    for j in range(D_MODEL // LANES):
                        frow = fbase + j * LANES
                        vals = plsc.load_gather(rows, [tok, frow])
                        plsc.store_scatter(
                            ob,
                            [rrv, jnp.right_shift(frow, 3), rrv * 0,
                             jnp.bitwise_and(frow, 7), tokl],
                            vals * SCALE,
                        )

        compute_p(0, 0)
        start_gather(0)
        compute_p(1, 1)

        @pl.loop(0, n_steps, step=NBUF)
        def _(g0):
            for b in range(NBUF):
                g = g0 + b
                nxt = g + 1

                @pl.when(g >= NBUF)
                def _():
                    wait_write(b)

                @pl.when(nxt < n_steps)
                def _():
                    start_gather((b + 1) % NBUF)

                wait_gather(b)
                compute_out(g, b)

                @pl.when(nxt + 1 < n_steps)
                def _():
                    compute_p(nxt + 1, b)

                start_write(g, b)

        wait_write((n_steps - 1) % NBUF)
        wait_write((n_steps - 2) % NBUF)

    return k


def kernel(x, lut):
    S0, S1 = x.shape
    V = lut.shape[0]
    n_quads = V // QB
    rem = V - n_quads * QB
    tail_rm = lut[V - rem:].reshape(rem * D_MODEL)
    tab = _build_relayout(V)(lut.T, tail_rm)
    out6 = _build_gather(S0, S1, V)(x.T, tab.reshape(V, D_MODEL))
    return out6.transpose(2, 4, 0, 1, 3).reshape(S0, S1, D_MODEL)


# speedup vs baseline: 1.1047x; 1.1047x over previous
import functools
import math

import jax
import jax.numpy as jnp
from jax import lax
from jax.experimental import pallas as pl
from jax.experimental.pallas import tpu as pltpu
from jax.experimental.pallas import tpu_sc as plsc

D_MODEL = 64
SCALE = math.sqrt(D_MODEL)

NUM_CORES = 2
NUM_SUBCORES = 16
LANES = 16
NW = NUM_CORES * NUM_SUBCORES

NBUF = 2


@functools.lru_cache(maxsize=None)
def _build_sc_gather(S0: int, S1: int, V: int):
    assert S0 % (NW * 128) == 0 or S0 == NW * 128
    lanes_per_w = S0 // NW

    mesh = plsc.VectorSubcoreMesh(core_axis_name="c", subcore_axis_name="s")

    @functools.partial(
        pl.kernel,
        out_type=jax.ShapeDtypeStruct((S1, D_MODEL, S0), jnp.float32),
        mesh=mesh,
        scratch_types=[
            pltpu.VMEM((S1, lanes_per_w), jnp.int32),
            pltpu.VMEM((NBUF, lanes_per_w), jnp.int32),
            pltpu.VMEM((NBUF, lanes_per_w, 128), jnp.float32),
            pltpu.VMEM((NBUF, D_MODEL, 130), jnp.float32),
            pltpu.SemaphoreType.DMA,
            pltpu.SemaphoreType.DMA,
        ],
        compiler_params=pltpu.CompilerParams(needs_layout_passes=False),
    )
    def k(xt_hbm, tab_hbm, out_hbm, idx_v, pb_v, rows_v, ob_v, gsem, wsem):
        wid = lax.axis_index("s") * NUM_CORES + lax.axis_index("c")
        base = wid * lanes_per_w
        pltpu.sync_copy(xt_hbm.at[:, pl.ds(base, lanes_per_w)], idx_v)

        def compute_p(g, slot):
            for kk in range(lanes_per_w // LANES):
                sl = pl.ds(kk * LANES, LANES)
                pb_v[slot, sl] = jnp.right_shift(idx_v[g, sl], 1)

        def start_gather(slot):
            pltpu.async_copy(tab_hbm.at[pb_v.at[slot]], rows_v.at[slot], gsem)

        def wait_gather(slot):
            pltpu.make_async_copy(
                tab_hbm.at[pb_v.at[slot]], rows_v.at[slot], gsem
            ).wait()

        def start_write(g, slot):
            pltpu.async_copy(
                ob_v.at[slot, :, pl.ds(0, lanes_per_w)],
                out_hbm.at[g, :, pl.ds(base, lanes_per_w)],
                wsem,
            )

        def wait_write(slot):
            pltpu.make_async_copy(
                ob_v.at[slot, :, pl.ds(0, lanes_per_w)],
                out_hbm.at[0, :, pl.ds(base, lanes_per_w)],
                wsem,
            ).wait()

        iota16 = jax.lax.iota(jnp.int32, LANES)

        def compute_out(g, slot):
            ob = ob_v.at[slot]
            rows = rows_v.at[slot]

            @pl.loop(0, lanes_per_w // LANES)
            def _(kk):
                tok = iota16 + kk * LANES
                hv = jnp.left_shift(
                    jnp.bitwise_and(idx_v[g, pl.ds(kk * LANES, LANES)], 1), 6
                )
                for d in range(LANES):
                    fbase = jnp.bitwise_and(iota16 + d, LANES - 1)
                    cbase = hv + fbase
                    for j in range(D_MODEL // LANES):
                        frow = fbase + j * LANES
                        vals = plsc.load_gather(rows, [tok, cbase + j * LANES])
                        plsc.store_scatter(ob, [frow, tok], vals * SCALE)

        assert S1 % NBUF == 0
        compute_p(0, 0)
        start_gather(0)
        compute_p(1, 1)

        @pl.loop(0, S1, step=NBUF)
        def _(g0):
            for b in range(NBUF):
                g = g0 + b
                nxt = g + 1

                @pl.when(g >= NBUF)
                def _():
                    wait_write(b)

                @pl.when(nxt < S1)
                def _():
                    start_gather((b + 1) % NBUF)

                wait_gather(b)
                compute_out(g, b)

                @pl.when(nxt + 1 < S1)
                def _():
                    compute_p(nxt + 1, b)

                start_write(g, b)

        wait_write(0)
        wait_write(1)

    return k


def kernel(x, lut):
    S0, S1 = x.shape
    V = lut.shape[0]
    tab = lut.reshape(V // 2, 2 * D_MODEL)
    k = _build_sc_gather(S0, S1, V)
    out = k(x.T, tab)
    return out.transpose(2, 0, 1)
